# baseline (speedup 1.0000x reference)
import jax
import jax.numpy as jnp
from jax import lax
from jax.experimental import pallas as pl
from jax.experimental.pallas import tpu as pltpu
from jax.experimental.pallas import tpu_sc as plsc

N = 10000
D = 128
O = 64
E = 320000

NC, NS, L = 2, 16, 16
NPAD = 10112
ROWS_T = 632
CH = 128
EPAD = 327680
EROWS = 2560
CHUNKS_DEG = 80
DUMMY = N

GDEG = 8
KPIPE = 2
PP = 16
CA = 80
CB = 80


def _deg_body(dst_hbm, zeros_hbm, ones_hbm, out_hbm, degtab, ones_v, idx_v,
              dsem):
    c = lax.axis_index("c")
    s = lax.axis_index("s")
    w = c * NS + s
    pltpu.sync_copy(zeros_hbm.at[pl.ds(ROWS_T * s, ROWS_T), :],
                    degtab.at[pl.ds(ROWS_T * s, ROWS_T), :])
    pltpu.sync_copy(ones_hbm, ones_v)
    pltpu.sync_copy(dst_hbm.at[pl.ds(CHUNKS_DEG * w, CHUNKS_DEG), :], idx_v)
    plsc.subcore_barrier()

    def body(i, carry):
        descs = [
            pltpu.async_copy(ones_v, degtab.at[idx_v.at[GDEG * i + g]],
                             dsem, add=True)
            for g in range(GDEG)
        ]
        for d in descs:
            d.wait()
        return carry

    lax.fori_loop(0, CHUNKS_DEG // GDEG, body, 0)
    plsc.subcore_barrier()
    pltpu.sync_copy(degtab.at[pl.ds(ROWS_T * s, ROWS_T), :],
                    out_hbm.at[c, pl.ds(ROWS_T * s, ROWS_T), :])


_deg_kernel = pl.kernel(
    _deg_body,
    out_type=jax.ShapeDtypeStruct((NC, NPAD, D), jnp.float32),
    mesh=plsc.VectorSubcoreMesh(core_axis_name="c", subcore_axis_name="s",
                                num_cores=NC, num_subcores=NS),
    scratch_types=[
        pltpu.VMEM_SHARED((NPAD, D), jnp.float32),
        pltpu.VMEM((CH, D), jnp.float32),
        pltpu.VMEM((CHUNKS_DEG, CH), jnp.int32),
        pltpu.SemaphoreType.DMA,
    ],
)


def _prop_body(y_hbm, src_hbm, dst_hbm, zeros_hbm, out_hbm,
               acctab, idx_s, idx_d, r0, r1, g0, g1, s0, s1):
    rows = [r0, r1]
    gsem = [g0, g1]
    ssem = [s0, s1]
    c = lax.axis_index("c")
    s = lax.axis_index("s")
    pltpu.sync_copy(zeros_hbm.at[pl.ds(ROWS_T * s, ROWS_T), :],
                    acctab.at[pl.ds(ROWS_T * s, ROWS_T), :])
    plsc.subcore_barrier()
    base = jnp.where(c == 0, CA * s, NS * CA + CB * s)
    nph = jnp.where(c == 0, CA // PP, CB // PP)

    def phase(p, carry):
        off = base + PP * p
        pltpu.sync_copy(src_hbm.at[pl.ds(off, PP), :], idx_s)
        pltpu.sync_copy(dst_hbm.at[pl.ds(off, PP), :], idx_d)

        for k in range(KPIPE):
            pltpu.async_copy(y_hbm.at[idx_s.at[k]], rows[k], gsem[k])
        for j0 in range(0, PP, KPIPE):
            sdescs = []
            for k in range(KPIPE):
                pltpu.make_async_copy(y_hbm.at[idx_s.at[j0 + k]], rows[k],
                                      gsem[k]).wait()
                sdescs.append(
                    pltpu.async_copy(rows[k], acctab.at[idx_d.at[j0 + k]],
                                     ssem[k], add=True))
            for k in range(KPIPE):
                sdescs[k].wait()
                jn = j0 + KPIPE + k
                if jn < PP:
                    pltpu.async_copy(y_hbm.at[idx_s.at[jn]], rows[k], gsem[k])
        return carry

    lax.fori_loop(0, nph, phase, 0)
    plsc.subcore_barrier()
    pltpu.sync_copy(acctab.at[pl.ds(ROWS_T * s, ROWS_T), :],
                    out_hbm.at[c, pl.ds(ROWS_T * s, ROWS_T), :])


_prop_kernel = pl.kernel(
    _prop_body,
    out_type=jax.ShapeDtypeStruct((NC, NPAD, D), jnp.float32),
    mesh=plsc.VectorSubcoreMesh(core_axis_name="c", subcore_axis_name="s",
                                num_cores=NC, num_subcores=NS),
    scratch_types=[
        pltpu.VMEM_SHARED((NPAD, D), jnp.float32),
        pltpu.VMEM((PP, CH), jnp.int32),
        pltpu.VMEM((PP, CH), jnp.int32),
    ] + [pltpu.VMEM((CH, D), jnp.float32)] * KPIPE
      + [pltpu.SemaphoreType.DMA] * (2 * KPIPE),
)


def _dinv(deg_ref):
    d = jnp.max(deg_ref[0], axis=-1) + jnp.max(deg_ref[1], axis=-1) + 1.0
    return lax.rsqrt(d)[:, None]


def _tc_a_body(x_ref, w_ref, y_ref):
    y_ref[...] = jnp.dot(x_ref[...], w_ref[...],
                         preferred_element_type=jnp.float32)


def _tc_scale_body(xw_ref, deg_ref, y_ref):
    y_ref[...] = xw_ref[...] * _dinv(deg_ref)


def _tc_b_body(acc_ref, y_ref, deg_ref, b_ref, w_ref, y2_ref):
    dinv = _dinv(deg_ref)
    acc = acc_ref[0] + acc_ref[1]
    h = jnp.maximum((acc + y_ref[...]) * dinv + b_ref[...], 0.0)
    y2_ref[...] = jnp.dot(h, w_ref[...],
                          preferred_element_type=jnp.float32) * dinv


def _tc_c_body(acc_ref, y_ref, deg_ref, b_ref, wc_ref, bc_ref, out_ref, ssum):
    i = pl.program_id(0)

    @pl.when(i == 0)
    def _():
        ssum[...] = jnp.zeros_like(ssum)

    dinv = _dinv(deg_ref)
    acc = acc_ref[0] + acc_ref[1]
    h = jnp.maximum((acc + y_ref[...]) * dinv + b_ref[...], 0.0)
    row = i * R + lax.broadcasted_iota(jnp.int32, (R, 1), 0)
    h = jnp.where(row < N, h, 0.0)
    ssum[...] += jnp.sum(h, axis=0, keepdims=True)

    @pl.when(i == pl.num_programs(0) - 1)
    def _():
        out_ref[...] = jnp.dot(ssum[...] * (1.0 / N), wc_ref[...],
                               preferred_element_type=jnp.float32) + bc_ref[...]


R = 1264

_ROW = pl.BlockSpec((R, D), lambda i: (i, 0))
_ACC = pl.BlockSpec((NC, R, D), lambda i: (0, i, 0))
_DEG_SPEC = pl.BlockSpec((NC, R, D), lambda i: (0, i, 0))
_FULL = lambda shape: pl.BlockSpec(shape, lambda i: (0,) * len(shape))

_Y_OUT = jax.ShapeDtypeStruct((NPAD, D), jnp.float32)

_tc_a = pl.pallas_call(
    _tc_a_body,
    grid=(NPAD // R,),
    in_specs=[_ROW, _FULL((D, D))],
    out_specs=_ROW,
    out_shape=_Y_OUT,
)

_tc_scale = pl.pallas_call(
    _tc_scale_body,
    grid=(NPAD // R,),
    in_specs=[_ROW, _DEG_SPEC],
    out_specs=_ROW,
    out_shape=_Y_OUT,
)

_tc_b = pl.pallas_call(
    _tc_b_body,
    grid=(NPAD // R,),
    in_specs=[_ACC, _ROW, _DEG_SPEC, _FULL((1, D)), _FULL((D, D))],
    out_specs=_ROW,
    out_shape=_Y_OUT,
)

_tc_c = pl.pallas_call(
    _tc_c_body,
    grid=(NPAD // R,),
    in_specs=[_ACC, _ROW, _DEG_SPEC,
              _FULL((1, D)), _FULL((D, O)), _FULL((1, O))],
    out_specs=_FULL((1, O)),
    out_shape=jax.ShapeDtypeStruct((1, O), jnp.float32),
    scratch_shapes=[pltpu.VMEM((1, D), jnp.float32)],
)


@jax.jit
def _run(x, edge_index, W1, b1, W2, b2, Wc, bc):
    src = edge_index[0].astype(jnp.int32)
    dst = edge_index[1].astype(jnp.int32)
    pad = jnp.full((EPAD - E,), DUMMY, dtype=jnp.int32)
    srcp = jnp.concatenate([src, pad]).reshape(EROWS, CH)
    dstp = jnp.concatenate([dst, pad]).reshape(EROWS, CH)

    xp = jnp.zeros((NPAD, D), jnp.float32).at[:N].set(x)
    ones_deg = jnp.ones((CH, D), jnp.float32)
    zeros_acc = jnp.zeros((NPAD, D), jnp.float32)

    degparts = _deg_kernel(dstp, zeros_acc, ones_deg)

    xw1 = _tc_a(xp, W1)
    y1 = _tc_scale(xw1, degparts)
    acc1 = _prop_kernel(y1, srcp, dstp, zeros_acc)
    y2 = _tc_b(acc1, y1, degparts, b1.reshape(1, D), W2)
    acc2 = _prop_kernel(y2, srcp, dstp, zeros_acc)
    out = _tc_c(acc2, y2, degparts, b2.reshape(1, D), Wc, bc.reshape(1, O))
    return out.reshape(O)


def kernel(x, edge_index, W1, b1, W2, b2, Wc, bc):
    return _run(x, edge_index, W1, b1, W2, b2, Wc, bc)


# speedup vs baseline: 1.1105x; 1.1105x over previous
import jax
import jax.numpy as jnp
from jax import lax
from jax.experimental import pallas as pl
from jax.experimental.pallas import tpu as pltpu
from jax.experimental.pallas import tpu_sc as plsc

N = 10000
D = 128
O = 64
E = 320000

NC, NS, L = 2, 16, 16
NPAD = 10112
ROWS_T = 632
CH = 128
EPAD = 327680
EROWS = 2560
CHUNKS_DEG = 80
DUMMY = N

GDEG = 8
KPIPE = 2
PP = 16
CA = 80
CB = 80


def _deg_body(dst_hbm, zeros_hbm, ones_hbm, out_hbm, degtab, ones_v, idx_v,
              dsem):
    c = lax.axis_index("c")
    s = lax.axis_index("s")
    w = c * NS + s
    pltpu.sync_copy(zeros_hbm.at[pl.ds(ROWS_T * s, ROWS_T), :],
                    degtab.at[pl.ds(ROWS_T * s, ROWS_T), :])
    pltpu.sync_copy(ones_hbm, ones_v)
    pltpu.sync_copy(dst_hbm.at[pl.ds(CHUNKS_DEG * w, CHUNKS_DEG), :], idx_v)
    plsc.subcore_barrier()

    def body(i, carry):
        descs = [
            pltpu.async_copy(ones_v, degtab.at[idx_v.at[GDEG * i + g]],
                             dsem, add=True)
            for g in range(GDEG)
        ]
        for d in descs:
            d.wait()
        return carry

    lax.fori_loop(0, CHUNKS_DEG // GDEG, body, 0)
    plsc.subcore_barrier()
    pltpu.sync_copy(degtab.at[pl.ds(ROWS_T * s, ROWS_T), :],
                    out_hbm.at[c, pl.ds(ROWS_T * s, ROWS_T), :])


_deg_kernel = pl.kernel(
    _deg_body,
    out_type=jax.ShapeDtypeStruct((NC, NPAD, D), jnp.float32),
    mesh=plsc.VectorSubcoreMesh(core_axis_name="c", subcore_axis_name="s",
                                num_cores=NC, num_subcores=NS),
    scratch_types=[
        pltpu.VMEM_SHARED((NPAD, D), jnp.float32),
        pltpu.VMEM((CH, D), jnp.float32),
        pltpu.VMEM((CHUNKS_DEG, CH), jnp.int32),
        pltpu.SemaphoreType.DMA,
    ],
)


def _prop_body(y_hbm, src_hbm, dst_hbm, zeros_hbm, out_hbm,
               acctab, idx_s, idx_d, r0, r1, g0, g1, s0, s1):
    rows = [r0, r1]
    gsem = [g0, g1]
    ssem = [s0, s1]
    c = lax.axis_index("c")
    s = lax.axis_index("s")
    pltpu.sync_copy(zeros_hbm.at[pl.ds(ROWS_T * s, ROWS_T), :],
                    acctab.at[pl.ds(ROWS_T * s, ROWS_T), :])
    plsc.subcore_barrier()
    base = jnp.where(c == 0, CA * s, NS * CA + CB * s)
    nph = jnp.where(c == 0, CA // PP, CB // PP)

    def phase(p, carry):
        off = base + PP * p
        pltpu.sync_copy(src_hbm.at[pl.ds(off, PP), :], idx_s)
        pltpu.sync_copy(dst_hbm.at[pl.ds(off, PP), :], idx_d)

        for k in range(KPIPE):
            pltpu.async_copy(y_hbm.at[idx_s.at[k]], rows[k], gsem[k])
        for j0 in range(0, PP, KPIPE):
            sdescs = []
            for k in range(KPIPE):
                pltpu.make_async_copy(y_hbm.at[idx_s.at[j0 + k]], rows[k],
                                      gsem[k]).wait()
                sdescs.append(
                    pltpu.async_copy(rows[k], acctab.at[idx_d.at[j0 + k]],
                                     ssem[k], add=True))
            for k in range(KPIPE):
                sdescs[k].wait()
                jn = j0 + KPIPE + k
                if jn < PP:
                    pltpu.async_copy(y_hbm.at[idx_s.at[jn]], rows[k], gsem[k])
        return carry

    lax.fori_loop(0, nph, phase, 0)
    plsc.subcore_barrier()
    pltpu.sync_copy(acctab.at[pl.ds(ROWS_T * s, ROWS_T), :],
                    out_hbm.at[c, pl.ds(ROWS_T * s, ROWS_T), :])


_prop_kernel = pl.kernel(
    _prop_body,
    out_type=jax.ShapeDtypeStruct((NC, NPAD, D), jnp.float32),
    mesh=plsc.VectorSubcoreMesh(core_axis_name="c", subcore_axis_name="s",
                                num_cores=NC, num_subcores=NS),
    scratch_types=[
        pltpu.VMEM_SHARED((NPAD, D), jnp.float32),
        pltpu.VMEM((PP, CH), jnp.int32),
        pltpu.VMEM((PP, CH), jnp.int32),
    ] + [pltpu.VMEM((CH, D), jnp.float32)] * KPIPE
      + [pltpu.SemaphoreType.DMA] * (2 * KPIPE),
)


def _dinv(deg_ref):
    d = jnp.max(deg_ref[0], axis=-1) + jnp.max(deg_ref[1], axis=-1) + 1.0
    return lax.rsqrt(d)[:, None]


def _tc_a_body(x_ref, w_ref, deg_ref, y_ref):
    y_ref[...] = jnp.dot(x_ref[...], w_ref[...],
                         preferred_element_type=jnp.float32) * _dinv(deg_ref)


def _tc_b_body(acc_ref, y_ref, deg_ref, b_ref, w_ref, y2_ref):
    dinv = _dinv(deg_ref)
    acc = acc_ref[0] + acc_ref[1]
    h = jnp.maximum((acc + y_ref[...]) * dinv + b_ref[...], 0.0)
    y2_ref[...] = jnp.dot(h, w_ref[...],
                          preferred_element_type=jnp.float32) * dinv


def _tc_c_body(acc_ref, y_ref, deg_ref, b_ref, wc_ref, bc_ref, out_ref, ssum):
    i = pl.program_id(0)

    @pl.when(i == 0)
    def _():
        ssum[...] = jnp.zeros_like(ssum)

    dinv = _dinv(deg_ref)
    acc = acc_ref[0] + acc_ref[1]
    h = jnp.maximum((acc + y_ref[...]) * dinv + b_ref[...], 0.0)
    row = i * R + lax.broadcasted_iota(jnp.int32, (R, 1), 0)
    h = jnp.where(row < N, h, 0.0)
    ssum[...] += jnp.sum(h, axis=0, keepdims=True)

    @pl.when(i == pl.num_programs(0) - 1)
    def _():
        out_ref[...] = jnp.dot(ssum[...] * (1.0 / N), wc_ref[...],
                               preferred_element_type=jnp.float32) + bc_ref[...]


R = 1264

_ROW = pl.BlockSpec((R, D), lambda i: (i, 0))
_ACC = pl.BlockSpec((NC, R, D), lambda i: (0, i, 0))
_DEG_SPEC = pl.BlockSpec((NC, R, D), lambda i: (0, i, 0))
_FULL = lambda shape: pl.BlockSpec(shape, lambda i: (0,) * len(shape))

_Y_OUT = jax.ShapeDtypeStruct((NPAD, D), jnp.float32)

_tc_a = pl.pallas_call(
    _tc_a_body,
    grid=(NPAD // R,),
    in_specs=[_ROW, _FULL((D, D)), _DEG_SPEC],
    out_specs=_ROW,
    out_shape=_Y_OUT,
)

_tc_b = pl.pallas_call(
    _tc_b_body,
    grid=(NPAD // R,),
    in_specs=[_ACC, _ROW, _DEG_SPEC, _FULL((1, D)), _FULL((D, D))],
    out_specs=_ROW,
    out_shape=_Y_OUT,
)

_tc_c = pl.pallas_call(
    _tc_c_body,
    grid=(NPAD // R,),
    in_specs=[_ACC, _ROW, _DEG_SPEC,
              _FULL((1, D)), _FULL((D, O)), _FULL((1, O))],
    out_specs=_FULL((1, O)),
    out_shape=jax.ShapeDtypeStruct((1, O), jnp.float32),
    scratch_shapes=[pltpu.VMEM((1, D), jnp.float32)],
)


@jax.jit
def _run(x, edge_index, W1, b1, W2, b2, Wc, bc):
    src = edge_index[0].astype(jnp.int32)
    dst = edge_index[1].astype(jnp.int32)
    pad = jnp.full((EPAD - E,), DUMMY, dtype=jnp.int32)
    srcp = jnp.concatenate([src, pad]).reshape(EROWS, CH)
    dstp = jnp.concatenate([dst, pad]).reshape(EROWS, CH)

    xp = jnp.zeros((NPAD, D), jnp.float32).at[:N].set(x)
    ones_deg = jnp.ones((CH, D), jnp.float32)
    zeros_acc = jnp.zeros((NPAD, D), jnp.float32)

    degparts = _deg_kernel(dstp, zeros_acc, ones_deg)

    y1 = _tc_a(xp, W1, degparts)
    acc1 = _prop_kernel(y1, srcp, dstp, zeros_acc)
    y2 = _tc_b(acc1, y1, degparts, b1.reshape(1, D), W2)
    acc2 = _prop_kernel(y2, srcp, dstp, zeros_acc)
    out = _tc_c(acc2, y2, degparts, b2.reshape(1, D), Wc, bc.reshape(1, O))
    return out.reshape(O)


def kernel(x, edge_index, W1, b1, W2, b2, Wc, bc):
    return _run(x, edge_index, W1, b1, W2, b2, Wc, bc)
